# baseline (speedup 1.0000x reference)
import functools

import jax
import jax.numpy as jnp
from jax.experimental import pallas as pl
from jax.experimental.pallas import tpu as pltpu

_N_NODES = 100000
_N_GRAPHS = 128
_D_NODE = 128
_D_EDGE = 16
_D_GLOBAL = 64
_HIDDEN = 128
_ROWS = 2000
_LN2 = 0.6931471805599453


def _tc_body(node_ref, sums_ref, cnt_ref, batch_ref, gf_ref, wnt_ref, wet_ref,
             wgt_ref, out_ref):
    h = jnp.dot(node_ref[...], wnt_ref[...], preferred_element_type=jnp.float32)
    mean = sums_ref[...] / jnp.maximum(cnt_ref[...], 1.0)
    h = h + jnp.dot(mean, wet_ref[...], preferred_element_type=jnp.float32)
    g = jnp.dot(gf_ref[...], wgt_ref[...], preferred_element_type=jnp.float32)
    iota = jax.lax.broadcasted_iota(jnp.int32, (_ROWS, _N_GRAPHS), 1)
    onehot = (batch_ref[...] == iota).astype(jnp.float32)
    h = h + jnp.dot(onehot, g, preferred_element_type=jnp.float32)
    out_ref[...] = (jnp.maximum(h, 0.0) + jnp.log1p(jnp.exp(-jnp.abs(h)))
                    - _LN2)


def _tc_call(node_feats, sums, cnt_col, batch_col, global_feats, wnt, wet, wgt):
    grid = (_N_NODES // _ROWS,)
    return pl.pallas_call(
        _tc_body,
        grid=grid,
        in_specs=[
            pl.BlockSpec((_ROWS, _D_NODE), lambda i: (i, 0)),
            pl.BlockSpec((_ROWS, _D_EDGE), lambda i: (i, 0)),
            pl.BlockSpec((_ROWS, 1), lambda i: (i, 0)),
            pl.BlockSpec((_ROWS, 1), lambda i: (i, 0)),
            pl.BlockSpec((_N_GRAPHS, _D_GLOBAL), lambda i: (0, 0)),
            pl.BlockSpec((_D_NODE, _HIDDEN), lambda i: (0, 0)),
            pl.BlockSpec((_D_EDGE, _HIDDEN), lambda i: (0, 0)),
            pl.BlockSpec((_D_GLOBAL, _HIDDEN), lambda i: (0, 0)),
        ],
        out_specs=pl.BlockSpec((_ROWS, _HIDDEN), lambda i: (i, 0)),
        out_shape=jax.ShapeDtypeStruct((_N_NODES, _HIDDEN), jnp.float32),
    )(node_feats, sums, cnt_col, batch_col, global_feats, wnt, wet, wgt)


def kernel(node_feats, edge_feats, global_feats, W, edge_index, batch):
    idx = edge_index[1]
    sums = jax.ops.segment_sum(edge_feats, idx, num_segments=_N_NODES)
    cnt = jax.ops.segment_sum(
        jnp.ones((edge_feats.shape[0],), edge_feats.dtype), idx,
        num_segments=_N_NODES)
    wnt = W[:, :_D_NODE].T
    wet = W[:, _D_NODE:_D_NODE + _D_EDGE].T
    wgt = W[:, _D_NODE + _D_EDGE:].T
    return _tc_call(node_feats, sums, cnt[:, None], batch[:, None],
                    global_feats, wnt, wet, wgt)


# speedup vs baseline: 3.5472x; 3.5472x over previous
import jax
import jax.numpy as jnp
from jax import lax
from jax.experimental import pallas as pl
from jax.experimental.pallas import tpu as pltpu
from jax.experimental.pallas import tpu_sc as plsc

_N_NODES = 100000
_N_PAD = 100352
_N_EDGES = 1600000
_N_GRAPHS = 128
_D_NODE = 128
_D_EDGE = 16
_D_GLOBAL = 64
_HIDDEN = 128
_ROWS = 2000
_LN2 = 0.6931471805599453

_NC, _NS = 2, 16
_NW = _NC * _NS
_CH = 125
_IDX_ROWS = _N_EDGES // _CH
_ROWS_PER_W = _IDX_ROWS // _NW
_KB = 8
_EPI = _KB * _CH
_ITERS = _ROWS_PER_W // _KB
_TILE_N = _N_PAD // _NS
_H = _N_PAD // 2
_CNT_SH = _H + 128


def _sc_body(idx_hbm, cidx_hbm, edges_hbm, acc_out, cnt_out,
             idx_v, cidx_v, rows_v, ones_v, lin_v, acc_sh, cnt_sh):
    cid = lax.axis_index("c")
    sid = lax.axis_index("s")
    wid = cid * _NS + sid
    base = sid * _TILE_N

    def _zero_rows(r, carry):
        rows_v[r, :] = jnp.zeros((16,), jnp.float32)
        return carry
    lax.fori_loop(0, 1024, _zero_rows, None)

    def _zero_lin(i, carry):
        lin_v[pl.ds(i * 16, 16)] = jnp.zeros((16,), jnp.float32)
        return carry
    lax.fori_loop(0, 6272 // 16, _zero_lin, None)

    def _ones(i, carry):
        ones_v[pl.ds(i * 16, 16)] = jnp.ones((16,), jnp.float32)
        return carry
    lax.fori_loop(0, 8, _ones, None)

    for k in range(6):
        pltpu.sync_copy(rows_v, acc_sh.at[pl.ds(base + k * 1024, 1024)])
    pltpu.sync_copy(rows_v.at[pl.ds(0, 128)],
                    acc_sh.at[pl.ds(base + 6144, 128)])
    pltpu.sync_copy(lin_v.at[pl.ds(0, 3072)],
                    cnt_sh.at[pl.ds(sid * 3072, 3072)])

    @pl.when(sid == 0)
    def _zero_cnt_tail():
        pltpu.sync_copy(lin_v.at[pl.ds(0, _CNT_SH - 49152)],
                        cnt_sh.at[pl.ds(49152, _CNT_SH - 49152)])

    plsc.subcore_barrier()

    def _step(g, carry):
        row0 = wid * _ROWS_PER_W + g * _KB
        ebase = row0 * _CH
        pltpu.sync_copy(idx_hbm.at[pl.ds(row0, _KB)], idx_v)
        pltpu.sync_copy(edges_hbm.at[pl.ds(ebase, _EPI)],
                        rows_v.at[pl.ds(0, _EPI)])
        for j in range(_KB):
            pltpu.sync_copy(rows_v.at[pl.ds(j * _CH, _CH)],
                            acc_sh.at[idx_v.at[j]], add=True)
        return carry
    lax.fori_loop(0, _ITERS, _step, None)

    def _cstep(g, carry):
        row0 = sid * (_IDX_ROWS // _NS) + g * _KB
        pltpu.sync_copy(cidx_hbm.at[cid].at[pl.ds(row0, _KB)], cidx_v)
        for j in range(_KB):
            pltpu.sync_copy(ones_v.at[pl.ds(0, _CH)],
                            cnt_sh.at[cidx_v.at[j]], add=True)
        return carry
    lax.fori_loop(0, _IDX_ROWS // _NS // _KB, _cstep, None)
    plsc.subcore_barrier()

    for k in range(7):
        n = 1024 if k < 6 else 128
        pltpu.sync_copy(acc_sh.at[pl.ds(base + k * 1024, n)],
                        rows_v.at[pl.ds(0, n)])
        pltpu.sync_copy(rows_v.at[pl.ds(0, n)],
                        acc_out.at[cid].at[pl.ds(base + k * 1024, n)])

    @pl.when(sid < 8)
    def _cnt_writeout():
        off = sid * 6272
        pltpu.sync_copy(cnt_sh.at[pl.ds(off, 6272)], lin_v)
        pltpu.sync_copy(lin_v, cnt_out.at[pl.ds(cid * _H + off, 6272)])


_sc_scatter = pl.kernel(
    _sc_body,
    mesh=plsc.VectorSubcoreMesh(core_axis_name="c", subcore_axis_name="s"),
    compiler_params=pltpu.CompilerParams(use_tc_tiling_on_sc=False),
    out_type=[
        jax.ShapeDtypeStruct((_NC, _N_PAD, _D_EDGE), jnp.float32),
        jax.ShapeDtypeStruct((_N_PAD,), jnp.float32),
    ],
    scratch_types=[
        pltpu.VMEM((_KB, _CH), jnp.int32),
        pltpu.VMEM((_KB, _CH), jnp.int32),
        pltpu.VMEM((1024, _D_EDGE), jnp.float32),
        pltpu.VMEM((128,), jnp.float32),
        pltpu.VMEM((6272,), jnp.float32),
        pltpu.VMEM_SHARED((_N_PAD, _D_EDGE), jnp.float32),
        pltpu.VMEM_SHARED((_CNT_SH,), jnp.float32),
    ],
)


def _tc_body(node_ref, acc_ref, cnt_ref, batch_ref, gf_ref, wnt_ref, wet_ref,
             wgt_ref, out_ref):
    h = jnp.dot(node_ref[...], wnt_ref[...], preferred_element_type=jnp.float32)
    sums = acc_ref[0] + acc_ref[1]
    mean = sums / jnp.maximum(cnt_ref[...], 1.0)
    h = h + jnp.dot(mean, wet_ref[...], preferred_element_type=jnp.float32)
    g = jnp.dot(gf_ref[...], wgt_ref[...], preferred_element_type=jnp.float32)
    iota = jax.lax.broadcasted_iota(jnp.int32, (_ROWS, _N_GRAPHS), 1)
    onehot = (batch_ref[...] == iota).astype(jnp.float32)
    h = h + jnp.dot(onehot, g, preferred_element_type=jnp.float32)
    out_ref[...] = (jnp.maximum(h, 0.0) + jnp.log1p(jnp.exp(-jnp.abs(h)))
                    - _LN2)


def _tc_call(node_feats, acc, cnt_col, batch_col, global_feats, wnt, wet, wgt):
    grid = (_N_NODES // _ROWS,)
    return pl.pallas_call(
        _tc_body,
        grid=grid,
        in_specs=[
            pl.BlockSpec((_ROWS, _D_NODE), lambda i: (i, 0)),
            pl.BlockSpec((_NC, _ROWS, _D_EDGE), lambda i: (0, i, 0)),
            pl.BlockSpec((_ROWS, 1), lambda i: (i, 0)),
            pl.BlockSpec((_ROWS, 1), lambda i: (i, 0)),
            pl.BlockSpec((_N_GRAPHS, _D_GLOBAL), lambda i: (0, 0)),
            pl.BlockSpec((_D_NODE, _HIDDEN), lambda i: (0, 0)),
            pl.BlockSpec((_D_EDGE, _HIDDEN), lambda i: (0, 0)),
            pl.BlockSpec((_D_GLOBAL, _HIDDEN), lambda i: (0, 0)),
        ],
        out_specs=pl.BlockSpec((_ROWS, _HIDDEN), lambda i: (i, 0)),
        out_shape=jax.ShapeDtypeStruct((_N_NODES, _HIDDEN), jnp.float32),
    )(node_feats, acc, cnt_col, batch_col, global_feats, wnt, wet, wgt)


def kernel(node_feats, edge_feats, global_feats, W, edge_index, batch):
    idx = edge_index[1]
    idx2d = idx.reshape(_IDX_ROWS, _CH)
    clo = jnp.where(idx < _H, idx, _H)
    chi = jnp.where(idx >= _H, idx - _H, _H)
    cidx = jnp.stack([clo, chi]).reshape(_NC, _IDX_ROWS, _CH)
    acc, cnt = _sc_scatter(idx2d, cidx, edge_feats)
    wnt = W[:, :_D_NODE].T
    wet = W[:, _D_NODE:_D_NODE + _D_EDGE].T
    wgt = W[:, _D_NODE + _D_EDGE:].T
    return _tc_call(node_feats, acc, cnt[:_N_NODES, None], batch[:, None],
                    global_feats, wnt, wet, wgt)


# speedup vs baseline: 3.8163x; 1.0759x over previous
import jax
import jax.numpy as jnp
from jax import lax
from jax.experimental import pallas as pl
from jax.experimental.pallas import tpu as pltpu
from jax.experimental.pallas import tpu_sc as plsc

_N_NODES = 100000
_N_PAD = 100352
_N_EDGES = 1600000
_N_GRAPHS = 128
_D_NODE = 128
_D_EDGE = 16
_D_GLOBAL = 64
_HIDDEN = 128
_ROWS = 2000
_LN2 = 0.6931471805599453

_NC, _NS = 2, 16
_NW = _NC * _NS
_CH = 125
_IDX_ROWS = _N_EDGES // _CH
_ROWS_PER_W = _IDX_ROWS // _NW
_EPW = _N_EDGES // _NW
_SUB = 500
_NSUB = _EPW // _SUB
_BODIES = _NSUB // 4
_CROWS_PER_TILE = _IDX_ROWS // _NS
_TILE_N = _N_PAD // _NS
_H = _N_PAD // 2
_CNT_SH = _H + 128


def _sc_body(idx_hbm, cidx_hbm, edges_hbm, acc_out, cnt_out,
             idx_v, cidx_v, rows_v, ones_v, lin_v, acc_sh, cnt_sh,
             ssem0, ssem1, csem0, csem1):
    cid = lax.axis_index("c")
    sid = lax.axis_index("s")
    wid = cid * _NS + sid
    base = sid * _TILE_N
    ssems = (ssem0, ssem1)
    csems = (csem0, csem1)

    def _zero_rows(r, carry):
        rows_v[0, r, :] = jnp.zeros((16,), jnp.float32)
        return carry
    lax.fori_loop(0, 512, _zero_rows, None)

    def _zero_lin(i, carry):
        lin_v[pl.ds(i * 16, 16)] = jnp.zeros((16,), jnp.float32)
        return carry
    lax.fori_loop(0, 3200 // 16, _zero_lin, None)

    def _ones(i, carry):
        ones_v[pl.ds(i * 16, 16)] = jnp.ones((16,), jnp.float32)
        return carry
    lax.fori_loop(0, 8, _ones, None)

    for k in range(12):
        pltpu.sync_copy(rows_v.at[0], acc_sh.at[pl.ds(base + k * 512, 512)])
    pltpu.sync_copy(rows_v.at[0].at[pl.ds(0, 128)],
                    acc_sh.at[pl.ds(base + 6144, 128)])
    pltpu.sync_copy(lin_v.at[pl.ds(0, 3072)],
                    cnt_sh.at[pl.ds(sid * 3072, 3072)])

    @pl.when(sid == 0)
    def _zero_cnt_tail():
        pltpu.sync_copy(lin_v.at[pl.ds(0, _CNT_SH - 49152)],
                        cnt_sh.at[pl.ds(49152, _CNT_SH - 49152)])

    plsc.subcore_barrier()

    def _body(gg, carry):
        for p in range(4):
            rb = p % 2
            ib = p // 2
            if p >= 2:
                pltpu.make_async_copy(
                    edges_hbm.at[pl.ds(0, _SUB)],
                    rows_v.at[rb].at[pl.ds(0, _SUB)], ssems[rb]).wait()
            else:
                @pl.when(gg >= 1)
                def _drain():
                    pltpu.make_async_copy(
                        edges_hbm.at[pl.ds(0, _SUB)],
                        rows_v.at[rb].at[pl.ds(0, _SUB)], ssems[rb]).wait()
            if p in (0, 2):
                irow0 = wid * _ROWS_PER_W + (gg * 2 + ib) * 8
                pltpu.sync_copy(idx_hbm.at[pl.ds(irow0, 8)], idx_v.at[ib])
                @pl.when(gg >= 1)
                def _cdrain():
                    pltpu.make_async_copy(
                        cnt_out.at[pl.ds(0, 2000)],
                        lin_v.at[pl.ds(0, 2000)], csems[ib]).wait()
                crow0 = sid * _CROWS_PER_TILE + gg * 32 + ib * 16
                pltpu.sync_copy(cidx_hbm.at[cid].at[pl.ds(crow0, 16)],
                                cidx_v.at[ib])
            s = gg * 4 + p
            pltpu.sync_copy(edges_hbm.at[pl.ds(wid * _EPW + s * _SUB, _SUB)],
                            rows_v.at[rb].at[pl.ds(0, _SUB)])
            for j in range(4):
                pltpu.async_copy(rows_v.at[rb].at[pl.ds(j * _CH, _CH)],
                                 acc_sh.at[idx_v.at[ib].at[rb * 4 + j]],
                                 ssems[rb], add=True)
            if p in (1, 3):
                cb = p // 2
                for j in range(16):
                    pltpu.async_copy(ones_v.at[pl.ds(0, _CH)],
                                     cnt_sh.at[cidx_v.at[cb].at[j]],
                                     csems[cb], add=True)
        return carry
    lax.fori_loop(0, _BODIES, _body, None)

    for rb in range(2):
        pltpu.make_async_copy(edges_hbm.at[pl.ds(0, _SUB)],
                              rows_v.at[rb].at[pl.ds(0, _SUB)],
                              ssems[rb]).wait()
    for cb in range(2):
        pltpu.make_async_copy(cnt_out.at[pl.ds(0, 2000)],
                              lin_v.at[pl.ds(0, 2000)], csems[cb]).wait()
    plsc.subcore_barrier()

    for k in range(13):
        n = 512 if k < 12 else 128
        b = k % 2
        if k >= 2:
            pltpu.make_async_copy(edges_hbm.at[pl.ds(0, 512)],
                                  rows_v.at[b], ssems[b]).wait()
        pltpu.sync_copy(acc_sh.at[pl.ds(base + k * 512, n)],
                        rows_v.at[b].at[pl.ds(0, n)])
        pltpu.async_copy(rows_v.at[b].at[pl.ds(0, n)],
                         acc_out.at[cid].at[pl.ds(base + k * 512, n)],
                         ssems[b])
    pltpu.make_async_copy(edges_hbm.at[pl.ds(0, 512)],
                          rows_v.at[1], ssems[1]).wait()
    pltpu.make_async_copy(edges_hbm.at[pl.ds(0, 128)],
                          rows_v.at[0].at[pl.ds(0, 128)], ssems[0]).wait()

    @pl.when(sid < 8)
    def _cnt_writeout():
        off = sid * 6272
        pltpu.sync_copy(cnt_sh.at[pl.ds(off, 3200)], lin_v)
        pltpu.sync_copy(lin_v, cnt_out.at[pl.ds(cid * _H + off, 3200)])
        pltpu.sync_copy(cnt_sh.at[pl.ds(off + 3200, 3072)],
                        lin_v.at[pl.ds(0, 3072)])
        pltpu.sync_copy(lin_v.at[pl.ds(0, 3072)],
                        cnt_out.at[pl.ds(cid * _H + off + 3200, 3072)])


_sc_scatter = pl.kernel(
    _sc_body,
    mesh=plsc.VectorSubcoreMesh(core_axis_name="c", subcore_axis_name="s"),
    compiler_params=pltpu.CompilerParams(use_tc_tiling_on_sc=False),
    out_type=[
        jax.ShapeDtypeStruct((_NC, _N_PAD, _D_EDGE), jnp.float32),
        jax.ShapeDtypeStruct((_N_PAD,), jnp.float32),
    ],
    scratch_types=[
        pltpu.VMEM((2, 8, _CH), jnp.int32),
        pltpu.VMEM((2, 16, _CH), jnp.int32),
        pltpu.VMEM((2, 512, _D_EDGE), jnp.float32),
        pltpu.VMEM((128,), jnp.float32),
        pltpu.VMEM((3200,), jnp.float32),
        pltpu.VMEM_SHARED((_N_PAD, _D_EDGE), jnp.float32),
        pltpu.VMEM_SHARED((_CNT_SH,), jnp.float32),
        pltpu.SemaphoreType.DMA,
        pltpu.SemaphoreType.DMA,
        pltpu.SemaphoreType.DMA,
        pltpu.SemaphoreType.DMA,
    ],
)


def _tc_body(node_ref, acc_ref, cnt_ref, batch_ref, gf_ref, wnt_ref, wet_ref,
             wgt_ref, out_ref):
    h = jnp.dot(node_ref[...], wnt_ref[...], preferred_element_type=jnp.float32)
    sums = acc_ref[0] + acc_ref[1]
    mean = sums / jnp.maximum(cnt_ref[...], 1.0)
    h = h + jnp.dot(mean, wet_ref[...], preferred_element_type=jnp.float32)
    g = jnp.dot(gf_ref[...], wgt_ref[...], preferred_element_type=jnp.float32)
    iota = jax.lax.broadcasted_iota(jnp.int32, (_ROWS, _N_GRAPHS), 1)
    onehot = (batch_ref[...] == iota).astype(jnp.float32)
    h = h + jnp.dot(onehot, g, preferred_element_type=jnp.float32)
    out_ref[...] = (jnp.maximum(h, 0.0) + jnp.log1p(jnp.exp(-jnp.abs(h)))
                    - _LN2)


def _tc_call(node_feats, acc, cnt_col, batch_col, global_feats, wnt, wet, wgt):
    grid = (_N_NODES // _ROWS,)
    return pl.pallas_call(
        _tc_body,
        grid=grid,
        in_specs=[
            pl.BlockSpec((_ROWS, _D_NODE), lambda i: (i, 0)),
            pl.BlockSpec((_NC, _ROWS, _D_EDGE), lambda i: (0, i, 0)),
            pl.BlockSpec((_ROWS, 1), lambda i: (i, 0)),
            pl.BlockSpec((_ROWS, 1), lambda i: (i, 0)),
            pl.BlockSpec((_N_GRAPHS, _D_GLOBAL), lambda i: (0, 0)),
            pl.BlockSpec((_D_NODE, _HIDDEN), lambda i: (0, 0)),
            pl.BlockSpec((_D_EDGE, _HIDDEN), lambda i: (0, 0)),
            pl.BlockSpec((_D_GLOBAL, _HIDDEN), lambda i: (0, 0)),
        ],
        out_specs=pl.BlockSpec((_ROWS, _HIDDEN), lambda i: (i, 0)),
        out_shape=jax.ShapeDtypeStruct((_N_NODES, _HIDDEN), jnp.float32),
    )(node_feats, acc, cnt_col, batch_col, global_feats, wnt, wet, wgt)


def kernel(node_feats, edge_feats, global_feats, W, edge_index, batch):
    idx = edge_index[1]
    idx2d = idx.reshape(_IDX_ROWS, _CH)
    clo = jnp.where(idx < _H, idx, _H)
    chi = jnp.where(idx >= _H, idx - _H, _H)
    cidx = jnp.stack([clo, chi]).reshape(_NC, _IDX_ROWS, _CH)
    acc, cnt = _sc_scatter(idx2d, cidx, edge_feats)
    wnt = W[:, :_D_NODE].T
    wet = W[:, _D_NODE:_D_NODE + _D_EDGE].T
    wgt = W[:, _D_NODE + _D_EDGE:].T
    return _tc_call(node_feats, acc, cnt[:_N_NODES, None], batch[:, None],
                    global_feats, wnt, wet, wgt)
